# Initial kernel scaffold; baseline (speedup 1.0000x reference)
import jax
import jax.numpy as jnp
from jax.experimental import pallas as pl


def kernel(z, codebook):
    raise NotImplementedError("write your pallas kernel here")


# speedup vs baseline: 1.0192x; 1.0192x over previous
import functools

import jax
import jax.numpy as jnp
from jax import lax
from jax.experimental import pallas as pl
from jax.experimental.pallas import tpu as pltpu
from jax.experimental.pallas import tpu_sc as plsc

_QT = 1024
_KT = 2048
_D = 32


def _round_bf16(x):
    u = lax.bitcast_convert_type(x, jnp.uint32)
    u = u + jnp.uint32(0x7FFF) + ((u >> jnp.uint32(16)) & jnp.uint32(1))
    u = u & jnp.uint32(0xFFFF0000)
    return lax.bitcast_convert_type(u, jnp.float32)


def _seq_row_sum(x):
    acc = x[0:1, :]
    for k in range(1, x.shape[0]):
        acc = acc + x[k:k + 1, :]
    return acc


def _seq_col_sum(x):
    acc = x[:, 0:1]
    for k in range(1, x.shape[1]):
        acc = acc + x[:, k:k + 1]
    return acc


def _argmin_body(q_ref, cbt_ref, idx_ref):
    q = q_ref[...]
    lhs = _round_bf16(2.0 * q)
    sum_q = _seq_col_sum(q * q)
    n_codes = cbt_ref.shape[1]

    best_val = jnp.full((q.shape[0], 1), jnp.inf, dtype=jnp.float32)
    best_idx = jnp.zeros((q.shape[0], 1), dtype=jnp.int32)
    for t in range(n_codes // _KT):
        cbt = cbt_ref[:, pl.ds(t * _KT, _KT)]
        cb_norm = _seq_row_sum(cbt * cbt)
        cross = lax.dot_general(
            lhs, _round_bf16(cbt), (((1,), (0,)), ((), ())),
            preferred_element_type=jnp.float32)
        scores = (sum_q - cross) + cb_norm
        local_min = jnp.min(scores, axis=1, keepdims=True)
        ids = lax.broadcasted_iota(jnp.int32, scores.shape, 1) + t * _KT
        local_arg = jnp.min(
            jnp.where(scores == local_min, ids, jnp.int32(2**30)),
            axis=1, keepdims=True)
        upd = local_min < best_val
        best_idx = jnp.where(upd, local_arg, best_idx)
        best_val = _round_bf16(jnp.where(upd, local_min, best_val))
    idx_ref[...] = best_idx


def _tc_argmin(flat, cbt):
    n, d = flat.shape
    return pl.pallas_call(
        _argmin_body,
        grid=(n // _QT,),
        in_specs=[
            pl.BlockSpec((_QT, d), lambda i: (i, 0)),
            pl.BlockSpec(cbt.shape, lambda i: (0, 0)),
        ],
        out_specs=pl.BlockSpec((_QT, 1), lambda i: (i, 0)),
        out_shape=jax.ShapeDtypeStruct((n, 1), jnp.int32),
        compiler_params=pltpu.CompilerParams(
            dimension_semantics=("parallel",)),
    )(flat, cbt)


def _sc_gather(table, idx):
    v, d = table.shape
    b = idx.shape[0]
    info = plsc.get_sparse_core_info()
    nw = info.num_cores * info.num_subcores
    b_per_w = b // nw
    mesh = plsc.VectorSubcoreMesh(core_axis_name="c", subcore_axis_name="s")

    @functools.partial(
        pl.kernel, mesh=mesh,
        compiler_params=pltpu.CompilerParams(use_tc_tiling_on_sc=False),
        out_type=jax.ShapeDtypeStruct((b, d), jnp.float32),
        scratch_types=[
            pltpu.VMEM((b_per_w,), jnp.int32),
            pltpu.VMEM((b_per_w, d), jnp.float32),
            pltpu.SemaphoreType.DMA,
        ],
    )
    def k(table_hbm, idx_hbm, out_hbm, idx_v, rows_v, sem):
        wid = lax.axis_index("s") * info.num_cores + lax.axis_index("c")
        base = wid * b_per_w
        pltpu.sync_copy(idx_hbm.at[pl.ds(base, b_per_w)], idx_v)
        pltpu.async_copy(table_hbm.at[idx_v], rows_v, sem).wait()
        pltpu.sync_copy(rows_v, out_hbm.at[pl.ds(base, b_per_w)])

    return k(table, idx)


def kernel(z, codebook):
    b, c, h, w = z.shape
    flat = jnp.transpose(z, (0, 2, 3, 1)).reshape(-1, c)
    cbt = jnp.transpose(codebook, (1, 0))
    idx = _tc_argmin(flat, cbt).reshape(-1)
    rows = _sc_gather(codebook, idx)
    out = jnp.transpose(rows.reshape(b, h, w, c), (0, 3, 1, 2))
    return (out, out)
